# baseline (speedup 1.0000x reference)
import functools
import math

import jax
import jax.numpy as jnp
from jax.experimental import pallas as pl

B, N, EMB = 4, 32, 16
K_BLK = 4

_HI = jax.lax.Precision.HIGHEST


def _sp(x):
    return jax.nn.softplus(5.0 * x) / 5.0


def _radial_mlp(w0, w1, w2, A1, bb1, A2, bb2):
    h = (w0 * A1[0][None, None, None, :] + w1 * A1[1][None, None, None, :]
         + w2 * A1[2][None, None, None, :] + bb1[None, None, None, :])
    h = _sp(h).reshape(B * N * N, 100)
    h = _sp(jnp.dot(h, A2, precision=_HI, preferred_element_type=jnp.float32) + bb2[None, :])
    return h


def _feat_body(features_ref, xyzT_ref, emb_ref, W1_ref, b1_ref, W2_ref, b2_ref,
               W3m_ref, V1_ref, c1_ref, V2_ref, c2_ref, Wres_ref, bres_ref,
               feat512_ref, hb_ref):
    feats = features_ref[...]
    emb = emb_ref[...]
    ff = (feats == 1).astype(jnp.float32)[..., None]
    f = emb[0][None, None, :] + ff * (emb[1] - emb[0])[None, None, :]

    x = xyzT_ref[0]
    y = xyzT_ref[1]
    z = xyzT_ref[2]
    dx = x[:, :, None] - x[:, None, :]
    dy = y[:, :, None] - y[:, None, :]
    dz = z[:, :, None] - z[:, None, :]
    dist = jnp.sqrt(dx * dx + dy * dy + dz * dz + 1e-12)
    nzf = (dist > 1e-5).astype(jnp.float32)
    inv = 1.0 / dist
    ux, uy, uz = dx * inv, dy * inv, dz * inv
    maskf = (dist < 1.4).astype(jnp.float32)
    nn_count = jnp.sum(maskf, -1)
    norm = 1.0 / jnp.sqrt(jnp.maximum(nn_count, 1.0))

    def basis(r):
        d = dist - r
        w = (jnp.cos((jnp.pi / 2.0) * d / 0.5) ** 2) * (jnp.abs(d) < 0.5).astype(jnp.float32)
        return w[..., None]

    w0, w1, w2 = basis(0.0), basis(0.5), basis(1.0)

    h = _radial_mlp(w0, w1, w2, W1_ref[...], b1_ref[...], W2_ref[...], b2_ref[...])
    hb = _radial_mlp(w0, w1, w2, V1_ref[...], c1_ref[...], V2_ref[...], c2_ref[...])
    h4 = h.reshape(B, N, N, 100)
    hb_ref[...] = hb.reshape(B, N, N, 100) * (1.0 / math.sqrt(float(N)))

    f2 = f.reshape(B * N, EMB)
    P = jax.lax.dot_general(f2, W3m_ref[...], (((1,), (1,)), ((), ())),
                            precision=_HI, preferred_element_type=jnp.float32)
    P4 = P.reshape(B, N, 100, 4 * EMB)

    h3 = h4.reshape(B * N, N, 100)
    P3 = P4.reshape(B * N, 100, 4 * EMB)
    Q = jax.lax.dot_general(h3, P3, (((2,), (1,)), ((0,), (0,))),
                            precision=_HI, preferred_element_type=jnp.float32)
    Qt = jnp.transpose(Q.reshape(B, N, N, 4 * EMB), (0, 2, 1, 3))

    mn = maskf * nzf
    Ys = [
        [0.28209479177387814 * maskf],
        [0.4886025119029199 * uy * mn,
         0.4886025119029199 * uz * mn,
         0.4886025119029199 * ux * mn],
        [1.0925484305920792 * ux * uy * mn,
         1.0925484305920792 * uy * uz * mn,
         0.31539156525252005 * (3.0 * uz * uz - 1.0) * mn,
         1.0925484305920792 * ux * uz * mn,
         0.5462742152960396 * (ux * ux - uy * uy) * mn],
        [0.5900435899266435 * uy * (3.0 * ux * ux - uy * uy) * mn,
         2.890611442640554 * ux * uy * uz * mn,
         0.4570457994644658 * uy * (5.0 * uz * uz - 1.0) * mn,
         0.3731763325901154 * uz * (5.0 * uz * uz - 3.0) * mn,
         0.4570457994644658 * ux * (5.0 * uz * uz - 1.0) * mn,
         1.445305721320277 * uz * (ux * ux - uy * uy) * mn,
         0.5900435899266435 * ux * (ux * ux - 3.0 * uy * uy) * mn],
    ]

    outs = []
    for l in range(4):
        Ql = Qt[..., l * EMB:(l + 1) * EMB]
        o_ms = [jnp.sum(S[..., None] * Ql, axis=2) for S in Ys[l]]
        o_l = jnp.stack(o_ms, axis=-1)
        outs.append(o_l.reshape(B, N, EMB * (2 * l + 1)))
    feat256 = jnp.concatenate(outs, -1) * norm[:, :, None]

    h2 = feat256.reshape(B * N, 256)
    for i in range(4):
        h2 = h2 + jax.nn.relu(
            jnp.dot(h2, Wres_ref[i], precision=_HI, preferred_element_type=jnp.float32)
            + bres_ref[i][None, :])
    feat512_ref[...] = jnp.concatenate([feat256, h2.reshape(B, N, 256)], -1)


def _coll_body(f512_ref, hbk_ref, v3_ref, out_ref):
    i = pl.program_id(0)

    @pl.when(i == 0)
    def _init():
        out_ref[...] = jnp.zeros_like(out_ref)

    fb = f512_ref[...]
    vb = v3_ref[...]
    T = jax.lax.dot_general(fb, vb, (((1,), (1,)), ((), ())),
                            preferred_element_type=jnp.float32)
    for zz in range(B):
        Tz = T[zz * N:(zz + 1) * N, :]
        acc = jnp.zeros((N, 512), jnp.float32)
        for kb in range(K_BLK):
            hz = hbk_ref[kb, zz * N:(zz + 1) * N, :]
            acc = acc + jnp.dot(hz, Tz[:, kb * 512:(kb + 1) * 512],
                                preferred_element_type=jnp.float32)
        out_ref[zz, :, :] = out_ref[zz, :, :] + acc


@jax.jit
def kernel(features, xyz, emb_table, W1, b1, W2, b2, W3, V1, c1, V2, c2, V3, Wres, bres):
    xyzT = jnp.transpose(xyz, (2, 0, 1))
    W3m = W3.reshape(100 * 4 * EMB, EMB)
    feat512, hb4 = pl.pallas_call(
        _feat_body,
        out_shape=[
            jax.ShapeDtypeStruct((B, N, 512), jnp.float32),
            jax.ShapeDtypeStruct((B, N, N, 100), jnp.float32),
        ],
    )(features, xyzT, emb_table, W1, b1, W2, b2, W3m, V1, c1, V2, c2, Wres, bres)

    hbk = jnp.transpose(hb4, (3, 0, 1, 2)).reshape(100, B * N, N)
    f512r = feat512.reshape(B * N, 512).astype(jnp.bfloat16)
    V3m = V3.reshape(100 * 512, 512).astype(jnp.bfloat16)
    coll = pl.pallas_call(
        _coll_body,
        grid=(100 // K_BLK,),
        in_specs=[
            pl.BlockSpec((B * N, 512), lambda i: (0, 0)),
            pl.BlockSpec((K_BLK, B * N, N), lambda i: (i, 0, 0)),
            pl.BlockSpec((K_BLK * 512, 512), lambda i: (i, 0)),
        ],
        out_specs=pl.BlockSpec((B, N, 512), lambda i: (0, 0, 0)),
        out_shape=jax.ShapeDtypeStruct((B, N, 512), jnp.float32),
    )(f512r, hbk, V3m)
    return (feat512, coll)


# speedup vs baseline: 1.7344x; 1.7344x over previous
import functools
import math

import jax
import jax.numpy as jnp
from jax.experimental import pallas as pl

B, N, EMB = 4, 32, 16
U_BLK = 16

_HI = jax.lax.Precision.HIGHEST


def _sp(x):
    return jax.nn.softplus(5.0 * x) / 5.0


def _radial_mlp(w0, w1, w2, A1, bb1, A2, bb2):
    h = (w0 * A1[0][None, None, None, :] + w1 * A1[1][None, None, None, :]
         + w2 * A1[2][None, None, None, :] + bb1[None, None, None, :])
    h = _sp(h).reshape(B * N * N, 100)
    h = _sp(jnp.dot(h, A2, precision=_HI, preferred_element_type=jnp.float32) + bb2[None, :])
    return h


def _feat_body(features_ref, xyzT_ref, emb_ref, W1_ref, b1_ref, W2_ref, b2_ref,
               W3m_ref, V1_ref, c1_ref, V2_ref, c2_ref, Wres_ref, bres_ref,
               feat512_ref, hb_ref):
    feats = features_ref[...]
    emb = emb_ref[...]
    ff = (feats == 1).astype(jnp.float32)[..., None]
    f = emb[0][None, None, :] + ff * (emb[1] - emb[0])[None, None, :]

    x = xyzT_ref[0]
    y = xyzT_ref[1]
    z = xyzT_ref[2]
    dx = x[:, :, None] - x[:, None, :]
    dy = y[:, :, None] - y[:, None, :]
    dz = z[:, :, None] - z[:, None, :]
    dist = jnp.sqrt(dx * dx + dy * dy + dz * dz + 1e-12)
    nzf = (dist > 1e-5).astype(jnp.float32)
    inv = 1.0 / dist
    ux, uy, uz = dx * inv, dy * inv, dz * inv
    maskf = (dist < 1.4).astype(jnp.float32)
    nn_count = jnp.sum(maskf, -1)
    norm = 1.0 / jnp.sqrt(jnp.maximum(nn_count, 1.0))

    def basis(r):
        d = dist - r
        w = (jnp.cos((jnp.pi / 2.0) * d / 0.5) ** 2) * (jnp.abs(d) < 0.5).astype(jnp.float32)
        return w[..., None]

    w0, w1, w2 = basis(0.0), basis(0.5), basis(1.0)

    h = _radial_mlp(w0, w1, w2, W1_ref[...], b1_ref[...], W2_ref[...], b2_ref[...])
    hb = _radial_mlp(w0, w1, w2, V1_ref[...], c1_ref[...], V2_ref[...], c2_ref[...])
    h4 = h.reshape(B, N, N, 100)
    hb_ref[...] = hb.reshape(B, N, N, 100) * (1.0 / math.sqrt(float(N)))

    f2 = f.reshape(B * N, EMB)
    P = jax.lax.dot_general(f2, W3m_ref[...], (((1,), (1,)), ((), ())),
                            precision=_HI, preferred_element_type=jnp.float32)
    P4 = P.reshape(B, N, 100, 4 * EMB)

    h3 = h4.reshape(B * N, N, 100)
    P3 = P4.reshape(B * N, 100, 4 * EMB)
    Q = jax.lax.dot_general(h3, P3, (((2,), (1,)), ((0,), (0,))),
                            precision=_HI, preferred_element_type=jnp.float32)
    Qt = jnp.transpose(Q.reshape(B, N, N, 4 * EMB), (0, 2, 1, 3))

    mn = maskf * nzf
    Ys = [
        [0.28209479177387814 * maskf],
        [0.4886025119029199 * uy * mn,
         0.4886025119029199 * uz * mn,
         0.4886025119029199 * ux * mn],
        [1.0925484305920792 * ux * uy * mn,
         1.0925484305920792 * uy * uz * mn,
         0.31539156525252005 * (3.0 * uz * uz - 1.0) * mn,
         1.0925484305920792 * ux * uz * mn,
         0.5462742152960396 * (ux * ux - uy * uy) * mn],
        [0.5900435899266435 * uy * (3.0 * ux * ux - uy * uy) * mn,
         2.890611442640554 * ux * uy * uz * mn,
         0.4570457994644658 * uy * (5.0 * uz * uz - 1.0) * mn,
         0.3731763325901154 * uz * (5.0 * uz * uz - 3.0) * mn,
         0.4570457994644658 * ux * (5.0 * uz * uz - 1.0) * mn,
         1.445305721320277 * uz * (ux * ux - uy * uy) * mn,
         0.5900435899266435 * ux * (ux * ux - 3.0 * uy * uy) * mn],
    ]

    outs = []
    for l in range(4):
        Ql = Qt[..., l * EMB:(l + 1) * EMB]
        o_ms = [jnp.sum(S[..., None] * Ql, axis=2) for S in Ys[l]]
        o_l = jnp.stack(o_ms, axis=-1)
        outs.append(o_l.reshape(B, N, EMB * (2 * l + 1)))
    feat256 = jnp.concatenate(outs, -1) * norm[:, :, None]

    h2 = feat256.reshape(B * N, 256)
    for i in range(4):
        h2 = h2 + jax.nn.relu(
            jnp.dot(h2, Wres_ref[i], precision=_HI, preferred_element_type=jnp.float32)
            + bres_ref[i][None, :])
    feat512_ref[...] = jnp.concatenate([feat256, h2.reshape(B, N, 256)], -1)


def _coll_body(f512_ref, hb3_ref, v3_ref, outT_ref):
    fb = f512_ref[...].astype(jnp.bfloat16)
    vb = v3_ref[...].reshape(100 * U_BLK, 512).astype(jnp.bfloat16)
    T = jax.lax.dot_general(fb, vb, (((1,), (1,)), ((), ())),
                            preferred_element_type=jnp.float32)
    T3 = T.reshape(B * N, 100, U_BLK)
    N2 = jax.lax.dot_general(hb3_ref[...], T3, (((2,), (1,)), ((0,), (0,))),
                             preferred_element_type=jnp.float32)
    N4 = N2.reshape(B, N, N, U_BLK)
    Cs = jnp.sum(N4, axis=1)
    outT_ref[...] = jnp.transpose(Cs, (2, 0, 1)).reshape(U_BLK, B * N)


@jax.jit
def kernel(features, xyz, emb_table, W1, b1, W2, b2, W3, V1, c1, V2, c2, V3, Wres, bres):
    xyzT = jnp.transpose(xyz, (2, 0, 1))
    W3m = W3.reshape(100 * 4 * EMB, EMB)
    feat512, hb4 = pl.pallas_call(
        _feat_body,
        out_shape=[
            jax.ShapeDtypeStruct((B, N, 512), jnp.float32),
            jax.ShapeDtypeStruct((B, N, N, 100), jnp.float32),
        ],
    )(features, xyzT, emb_table, W1, b1, W2, b2, W3m, V1, c1, V2, c2, Wres, bres)

    hb3 = hb4.reshape(B * N, N, 100)
    f512r = feat512.reshape(B * N, 512)
    collT = pl.pallas_call(
        _coll_body,
        grid=(512 // U_BLK,),
        in_specs=[
            pl.BlockSpec((B * N, 512), lambda j: (0, 0)),
            pl.BlockSpec((B * N, N, 100), lambda j: (0, 0, 0)),
            pl.BlockSpec((100, U_BLK * 512), lambda j: (0, j)),
        ],
        out_specs=pl.BlockSpec((U_BLK, B * N), lambda j: (j, 0)),
        out_shape=jax.ShapeDtypeStruct((512, B * N), jnp.float32),
    )(f512r, hb3, V3)
    coll = jnp.transpose(collT).reshape(B, N, 512)
    return (feat512, coll)
